# Initial kernel scaffold; baseline (speedup 1.0000x reference)
import jax
import jax.numpy as jnp
from jax.experimental import pallas as pl


def kernel(mesh_signal, bary_coordinates, template_weights, bias):
    raise NotImplementedError("write your pallas kernel here")


# speedup vs baseline: 7.7465x; 7.7465x over previous
import functools

import jax
import jax.numpy as jnp
from jax import lax
from jax.experimental import pallas as pl
from jax.experimental.pallas import tpu as pltpu
from jax.experimental.pallas import tpu_sc as plsc

N, R, A, F, T = 10000, 5, 8, 64, 16
NRA = N * R * A
NC, NS = 2, 16
NW = NC * NS
CHUNK = 400
NCHUNKS = NRA // CHUNK
CPW = (NCHUNKS + NW - 1) // NW


def _sc_interp(table, i0, i1, i2, w0a, w1a, w2a):
    mesh = plsc.VectorSubcoreMesh(core_axis_name="c", subcore_axis_name="s")

    @functools.partial(
        pl.kernel,
        out_type=jax.ShapeDtypeStruct((NRA, F), jnp.float32),
        mesh=mesh,
        compiler_params=pltpu.CompilerParams(use_tc_tiling_on_sc=False),
        scratch_types=[
            pltpu.VMEM((CHUNK,), jnp.int32),
            pltpu.VMEM((CHUNK,), jnp.int32),
            pltpu.VMEM((CHUNK,), jnp.int32),
            pltpu.VMEM((CHUNK,), jnp.float32),
            pltpu.VMEM((CHUNK,), jnp.float32),
            pltpu.VMEM((CHUNK,), jnp.float32),
            pltpu.VMEM((CHUNK, F), jnp.float32),
            pltpu.VMEM((CHUNK, F), jnp.float32),
            pltpu.VMEM((CHUNK, F), jnp.float32),
            pltpu.VMEM((CHUNK, F), jnp.float32),
            pltpu.SemaphoreType.DMA,
        ],
    )
    def k(table_hbm, i0_hbm, i1_hbm, i2_hbm, w0_hbm, w1_hbm, w2_hbm, out_hbm,
          x0, x1, x2, wv0r, wv1r, wv2r, r0, r1, r2, acc, sem):
        wid = lax.axis_index("s") * NC + lax.axis_index("c")

        def chunk_body(kk, carry):
            ci = wid + kk * NW

            @pl.when(ci < NCHUNKS)
            def _():
                base = ci * CHUNK
                sl_in = pl.ds(base, CHUNK)
                pltpu.sync_copy(i0_hbm.at[sl_in], x0)
                pltpu.sync_copy(i1_hbm.at[sl_in], x1)
                pltpu.sync_copy(i2_hbm.at[sl_in], x2)
                pltpu.sync_copy(w0_hbm.at[sl_in], wv0r)
                pltpu.sync_copy(w1_hbm.at[sl_in], wv1r)
                pltpu.sync_copy(w2_hbm.at[sl_in], wv2r)
                copies = [
                    pltpu.async_copy(table_hbm.at[x0], r0, sem),
                    pltpu.async_copy(table_hbm.at[x1], r1, sem),
                    pltpu.async_copy(table_hbm.at[x2], r2, sem),
                ]
                for cp in copies:
                    cp.wait()

                def group_body(g, c2):
                    gb = g * 16
                    wv0 = wv0r[pl.ds(gb, 16)]
                    wv1 = wv1r[pl.ds(gb, 16)]
                    wv2 = wv2r[pl.ds(gb, 16)]
                    for j in range(16):
                        i = gb + j
                        w0 = wv0[j]
                        w1 = wv1[j]
                        w2 = wv2[j]
                        for cb in range(F // 16):
                            sl = pl.ds(cb * 16, 16)
                            acc[i, sl] = (w0 * r0[i, sl] + w1 * r1[i, sl]
                                          + w2 * r2[i, sl])
                    return c2

                lax.fori_loop(0, CHUNK // 16, group_body, 0)
                pltpu.sync_copy(acc, out_hbm.at[pl.ds(base, CHUNK)])

            return carry

        lax.fori_loop(0, CPW, chunk_body, 0)

    return k(table, i0, i1, i2, w0a, w1a, w2a)


def _tc_matmul_bias_relu(a, b, bias_row):
    n, kdim = a.shape
    m = b.shape[1]
    bn = 1000

    def mmk(a_ref, b_ref, bias_ref, o_ref):
        o = jnp.dot(a_ref[...], b_ref[...], preferred_element_type=jnp.float32)
        o_ref[...] = jnp.maximum(o + bias_ref[...], 0.0)

    return pl.pallas_call(
        mmk,
        grid=(n // bn,),
        in_specs=[
            pl.BlockSpec((bn, kdim), lambda i: (i, 0)),
            pl.BlockSpec((kdim, m), lambda i: (0, 0)),
            pl.BlockSpec((1, m), lambda i: (0, 0)),
        ],
        out_specs=pl.BlockSpec((bn, m), lambda i: (i, 0)),
        out_shape=jax.ShapeDtypeStruct((n, m), jnp.float32),
    )(a, b, bias_row)


def kernel(mesh_signal, bary_coordinates, template_weights, bias):
    idx = bary_coordinates[..., 0].astype(jnp.int32)
    w = bary_coordinates[..., 1]
    idx3 = jnp.moveaxis(idx, -1, 0).reshape(3, NRA)
    w3 = jnp.moveaxis(w, -1, 0).reshape(3, NRA)

    interp = _sc_interp(mesh_signal, idx3[0], idx3[1], idx3[2],
                        w3[0], w3[1], w3[2])

    tw0 = template_weights[:, 0].reshape(T, F, R, A)
    rot_idx = (jnp.arange(A)[None, :] + jnp.arange(A)[:, None]) % A
    twr = tw0[:, :, :, rot_idx]
    wbig = jnp.transpose(twr, (2, 4, 1, 3, 0)).reshape(R * A * F, A * T)
    bias_row = jnp.tile(bias[:, 0], A)[None, :]

    out = _tc_matmul_bias_relu(interp.reshape(N, R * A * F), wbig, bias_row)
    return out.reshape(N, A, T)
